# baseline (speedup 1.0000x reference)
import functools

import jax
import jax.numpy as jnp
from jax import lax
from jax.experimental import pallas as pl
from jax.experimental.pallas import tpu as pltpu
from jax.experimental.pallas import tpu_sc as plsc

_NC = 2
_NS = 16
_NW = _NC * _NS
_K = 16
_NBUF = 4


def _build(b_flat, d):
    rows_per_w = b_flat // _NW
    nchunks = rows_per_w // _K
    ngroups = nchunks // _NBUF
    mesh = plsc.VectorSubcoreMesh(core_axis_name="c", subcore_axis_name="s")

    @functools.partial(
        pl.kernel,
        mesh=mesh,
        out_type=jax.ShapeDtypeStruct((b_flat, d), jnp.float32),
        scratch_types=[
            pltpu.VMEM((nchunks, _K), jnp.int32),
        ]
        + [pltpu.VMEM((_K, d), jnp.float32)] * _NBUF
        + [pltpu.SemaphoreType.DMA] * (2 * _NBUF),
    )
    def k(ids_hbm, table_hbm, out_hbm, idx_v, *scratch):
        bufs = scratch[:_NBUF]
        gsems = scratch[_NBUF : 2 * _NBUF]
        wsems = scratch[2 * _NBUF :]
        wid = lax.axis_index("s") * _NC + lax.axis_index("c")
        base = wid * rows_per_w
        pltpu.sync_copy(ids_hbm.at[wid], idx_v)

        pltpu.async_copy(table_hbm.at[idx_v.at[0]], bufs[0], gsems[0])
        pltpu.async_copy(table_hbm.at[idx_v.at[1]], bufs[1], gsems[1])

        def step(p, b):
            c = p * _NBUF + b
            pltpu.make_async_copy(
                table_hbm.at[idx_v.at[c]], bufs[b], gsems[b]
            ).wait()
            pltpu.async_copy(
                bufs[b], out_hbm.at[pl.ds(base + c * _K, _K)], wsems[b]
            )
            t = c + 2
            bt = (b + 2) % _NBUF

            @pl.when(t < nchunks)
            def _():
                @pl.when(t >= _NBUF)
                def _():
                    pltpu.make_async_copy(
                        bufs[bt],
                        out_hbm.at[pl.ds(base + (t - _NBUF) * _K, _K)],
                        wsems[bt],
                    ).wait()

                pltpu.async_copy(table_hbm.at[idx_v.at[t]], bufs[bt], gsems[bt])

        def body(p, carry):
            for b in range(_NBUF):
                step(p, b)
            return carry

        lax.fori_loop(0, ngroups, body, 0)

        for b in range(_NBUF):
            c = nchunks - _NBUF + b
            pltpu.make_async_copy(
                bufs[b], out_hbm.at[pl.ds(base + c * _K, _K)], wsems[b]
            ).wait()

    return k


def kernel(input_ids, embedding_weight):
    bt, s = input_ids.shape
    b_flat = bt * s
    d = embedding_weight.shape[1]
    ids3 = input_ids.reshape(_NW, b_flat // _NW // _K, _K).astype(jnp.int32)
    out = _build(b_flat, d)(ids3, embedding_weight)
    return out.reshape(bt, s, d)


# speedup vs baseline: 1.0056x; 1.0056x over previous
import functools

import jax
import jax.numpy as jnp
from jax import lax
from jax.experimental import pallas as pl
from jax.experimental.pallas import tpu as pltpu
from jax.experimental.pallas import tpu_sc as plsc

_NC = 2
_NS = 16
_NW = _NC * _NS
_K = 16
_NBUF = 4


def _build(b_flat, d):
    rows_per_w = b_flat // _NW
    nchunks = rows_per_w // _K
    ngroups = nchunks // _NBUF
    mesh = plsc.VectorSubcoreMesh(core_axis_name="c", subcore_axis_name="s")

    @functools.partial(
        pl.kernel,
        mesh=mesh,
        out_type=jax.ShapeDtypeStruct((b_flat, d), jnp.float32),
        scratch_types=[
            pltpu.VMEM((rows_per_w,), jnp.int32),
        ]
        + [pltpu.VMEM((_K, d), jnp.float32)] * _NBUF
        + [pltpu.SemaphoreType.DMA] * (2 * _NBUF),
    )
    def k(ids_hbm, table_hbm, out_hbm, idx_v, *scratch):
        bufs = scratch[:_NBUF]
        gsems = scratch[_NBUF : 2 * _NBUF]
        wsems = scratch[2 * _NBUF :]
        wid = lax.axis_index("s") * _NC + lax.axis_index("c")
        base = wid * rows_per_w
        w_per_row = ids_hbm.shape[1] // rows_per_w
        pltpu.sync_copy(
            ids_hbm.at[wid // w_per_row,
                       pl.ds((wid % w_per_row) * rows_per_w, rows_per_w)],
            idx_v,
        )

        pltpu.async_copy(table_hbm.at[idx_v.at[pl.ds(0, _K)]], bufs[0], gsems[0])
        pltpu.async_copy(table_hbm.at[idx_v.at[pl.ds(_K, _K)]], bufs[1], gsems[1])

        def step(p, b):
            c = p * _NBUF + b
            pltpu.make_async_copy(
                table_hbm.at[idx_v.at[pl.ds(c * _K, _K)]], bufs[b], gsems[b]
            ).wait()
            pltpu.async_copy(
                bufs[b], out_hbm.at[pl.ds(base + c * _K, _K)], wsems[b]
            )
            t = c + 2
            bt = (b + 2) % _NBUF

            @pl.when(t < nchunks)
            def _():
                @pl.when(t >= _NBUF)
                def _():
                    pltpu.make_async_copy(
                        bufs[bt],
                        out_hbm.at[pl.ds(base + (t - _NBUF) * _K, _K)],
                        wsems[bt],
                    ).wait()

                pltpu.async_copy(table_hbm.at[idx_v.at[pl.ds(t * _K, _K)]], bufs[bt], gsems[bt])

        def body(p, carry):
            for b in range(_NBUF):
                step(p, b)
            return carry

        lax.fori_loop(0, ngroups, body, 0)

        for b in range(_NBUF):
            c = nchunks - _NBUF + b
            pltpu.make_async_copy(
                bufs[b], out_hbm.at[pl.ds(base + c * _K, _K)], wsems[b]
            ).wait()

    return k


def kernel(input_ids, embedding_weight):
    bt, s = input_ids.shape
    b_flat = bt * s
    d = embedding_weight.shape[1]
    out = _build(b_flat, d)(input_ids, embedding_weight)
    return out.reshape(bt, s, d)
